# Initial kernel scaffold; baseline (speedup 1.0000x reference)
import functools
import math

import jax
import jax.numpy as jnp
from jax import lax
from jax.experimental import pallas as pl
from jax.experimental.pallas import tpu as pltpu
from jax.experimental.pallas import tpu_sc as plsc

N = 131072
D = 128
C = 8
LANES = 16
NC = 2
NS = 16
NW = NC * NS
G1 = N // C
G2 = G1 // C
ROWS_W = N // NW
OUT_W = G2 // NW
CHUNK_G = 16
CHUNK_ROWS = CHUNK_G * C
NCHUNK = (G1 // NW) // CHUNK_G
KV = D // LANES
INV_SQRT_D = 1.0 / math.sqrt(D)
NEG = -1e30


def _attend(src, row0, q_buf, w_buf, dst, drow):
    qv = [q_buf[pl.ds(k * LANES, LANES)] for k in range(KV)]
    lanes = lax.broadcasted_iota(jnp.int32, (LANES,), 0)
    sv = jnp.full((LANES,), NEG, jnp.float32)
    for i in range(C):
        p = src[row0 + i, pl.ds(0, LANES)] * qv[0]
        for k in range(1, KV):
            p = p + src[row0 + i, pl.ds(k * LANES, LANES)] * qv[k]
        s = jnp.sum(p) * INV_SQRT_D
        sv = jnp.where(lanes == i, s, sv)
    m = jnp.max(sv)
    e = jnp.exp(sv - m)
    w_buf[...] = e / jnp.sum(e)
    ws = [w_buf[i] for i in range(C)]
    for k in range(KV):
        sl = pl.ds(k * LANES, LANES)
        acc = ws[0] * src[row0, sl]
        for i in range(1, C):
            acc = acc + ws[i] * src[row0 + i, sl]
        dst[drow, sl] = acc


@functools.partial(
    pl.kernel,
    out_type=jax.ShapeDtypeStruct((G2, D), jnp.float32),
    mesh=plsc.VectorSubcoreMesh(
        core_axis_name="c", subcore_axis_name="s", num_cores=NC, num_subcores=NS
    ),
    scratch_types=[
        pltpu.VMEM((CHUNK_ROWS, D), jnp.float32),
        pltpu.VMEM((CHUNK_G, D), jnp.float32),
        pltpu.VMEM((OUT_W, D), jnp.float32),
        pltpu.VMEM((D,), jnp.float32),
        pltpu.VMEM((LANES,), jnp.float32),
    ],
)
def _sc_kernel(x_hbm, q_hbm, out_hbm, in_buf, y_buf, out_buf, q_buf, w_buf):
    wid = lax.axis_index("s") * NC + lax.axis_index("c")
    pltpu.sync_copy(q_hbm, q_buf)

    def chunk_body(cidx, carry):
        base = wid * ROWS_W + cidx * CHUNK_ROWS
        pltpu.sync_copy(x_hbm.at[pl.ds(base, CHUNK_ROWS), :], in_buf)

        def group_body(g, carry2):
            _attend(in_buf, g * C, q_buf, w_buf, y_buf, g)
            return carry2

        lax.fori_loop(0, CHUNK_G, group_body, 0)
        for j in range(CHUNK_G // C):
            _attend(y_buf, j * C, q_buf, w_buf, out_buf, cidx * (CHUNK_G // C) + j)
        return carry

    lax.fori_loop(0, NCHUNK, chunk_body, 0)
    pltpu.sync_copy(out_buf, out_hbm.at[pl.ds(wid * OUT_W, OUT_W)])


def kernel(x, segment_ids, q):
    del segment_ids
    return _sc_kernel(x, q)


# speedup vs baseline: 1.3870x; 1.3870x over previous
import functools
import math

import jax
import jax.numpy as jnp
from jax import lax
from jax.experimental import pallas as pl
from jax.experimental.pallas import tpu as pltpu
from jax.experimental.pallas import tpu_sc as plsc

N = 131072
D = 128
C = 8
LANES = 16
NC = 2
NS = 16
NW = NC * NS
G1 = N // C
G2 = G1 // C
ROWS_W = N // NW
OUT_W = G2 // NW
CHUNK_G = 16
CHUNK_ROWS = CHUNK_G * C
NCHUNK = (G1 // NW) // CHUNK_G
KV = D // LANES
INV_SQRT_D = 1.0 / math.sqrt(D)
NEG = -1e30


def _attend(src, row0, q_buf, dst, drow):
    qv = [q_buf[pl.ds(k * LANES, LANES)] for k in range(KV)]
    lanes = lax.broadcasted_iota(jnp.int32, (LANES,), 0)
    sv = jnp.full((LANES,), NEG, jnp.float32)
    for i in range(C):
        p = src[row0 + i, pl.ds(0, LANES)] * qv[0]
        for k in range(1, KV):
            p = p + src[row0 + i, pl.ds(k * LANES, LANES)] * qv[k]
        s = jnp.sum(p) * INV_SQRT_D
        sv = jnp.where(lanes == i, s, sv)
    m = jnp.max(sv)
    e = jnp.exp(sv - m)
    w = e / jnp.sum(e)
    ws = [w[i] for i in range(C)]
    for k in range(KV):
        sl = pl.ds(k * LANES, LANES)
        acc = ws[0] * src[row0, sl]
        for i in range(1, C):
            acc = acc + ws[i] * src[row0 + i, sl]
        dst[drow, sl] = acc


@functools.partial(
    pl.kernel,
    out_type=jax.ShapeDtypeStruct((G2, D), jnp.float32),
    mesh=plsc.VectorSubcoreMesh(
        core_axis_name="c", subcore_axis_name="s", num_cores=NC, num_subcores=NS
    ),
    scratch_types=[
        pltpu.VMEM((CHUNK_ROWS, D), jnp.float32),
        pltpu.VMEM((CHUNK_G, D), jnp.float32),
        pltpu.VMEM((OUT_W, D), jnp.float32),
        pltpu.VMEM((D,), jnp.float32),
    ],
    compiler_params=pltpu.CompilerParams(needs_layout_passes=False),
)
def _sc_kernel(x_hbm, q_hbm, out_hbm, in_buf, y_buf, out_buf, q_buf):
    wid = lax.axis_index("s") * NC + lax.axis_index("c")
    pltpu.sync_copy(q_hbm, q_buf)

    def chunk_body(cidx, carry):
        base = wid * ROWS_W + cidx * CHUNK_ROWS
        pltpu.sync_copy(x_hbm.at[pl.ds(base, CHUNK_ROWS), :], in_buf)

        def group_body(g, carry2):
            _attend(in_buf, g * C, q_buf, y_buf, g)
            return carry2

        lax.fori_loop(0, CHUNK_G, group_body, 0)
        for j in range(CHUNK_G // C):
            _attend(y_buf, j * C, q_buf, out_buf, cidx * (CHUNK_G // C) + j)
        return carry

    lax.fori_loop(0, NCHUNK, chunk_body, 0)
    pltpu.sync_copy(out_buf, out_hbm.at[pl.ds(wid * OUT_W, OUT_W)])


def kernel(x, segment_ids, q):
    del segment_ids
    return _sc_kernel(x, q)


# speedup vs baseline: 1.8918x; 1.3640x over previous
import functools
import math

import jax
import jax.numpy as jnp
from jax import lax
from jax.experimental import pallas as pl
from jax.experimental.pallas import tpu as pltpu
from jax.experimental.pallas import tpu_sc as plsc

N = 131072
D = 128
C = 8
LANES = 16
NC = 2
NS = 16
NW = NC * NS
G1 = N // C
G2 = G1 // C
ROWS_W = N // NW
OUT_W = G2 // NW
CHUNK_G = 16
CHUNK_ROWS = CHUNK_G * C
NCHUNK = (G1 // NW) // CHUNK_G
KV = D // LANES
INV_SQRT_D = 1.0 / math.sqrt(D)


def _bcast(v, lane):
    return v.at[jnp.full((LANES,), lane, jnp.int32)].get(mode="promise_in_bounds")


def _attend2(src, row0, qv, dst, drow):
    idx = lax.broadcasted_iota(jnp.int32, (LANES,), 0)
    sv = jnp.zeros((LANES,), jnp.float32)
    for i in range(2 * C):
        r = row0 + i
        m = [src[r, pl.ds(k * LANES, LANES)] * qv[k] for k in range(KV)]
        t = ((m[0] + m[1]) + (m[2] + m[3])) + ((m[4] + m[5]) + (m[6] + m[7]))
        for sh in (1, 2, 4, 8):
            t = t + t.at[idx ^ sh].get(mode="promise_in_bounds")
        sv = jnp.where(idx == i, t, sv)
    sv = sv * INV_SQRT_D
    mx = sv
    for sh in (1, 2, 4):
        mx = jnp.maximum(mx, mx.at[idx ^ sh].get(mode="promise_in_bounds"))
    e = jnp.exp(sv - mx)
    z = e
    for sh in (1, 2, 4):
        z = z + z.at[idx ^ sh].get(mode="promise_in_bounds")
    w = e / z
    wb = [_bcast(w, i) for i in range(2 * C)]
    for gg in range(2):
        for k in range(KV):
            sl = pl.ds(k * LANES, LANES)
            acc = wb[gg * C] * src[row0 + gg * C, sl]
            for i in range(1, C):
                acc = acc + wb[gg * C + i] * src[row0 + gg * C + i, sl]
            dst[drow + gg, sl] = acc


@functools.partial(
    pl.kernel,
    out_type=jax.ShapeDtypeStruct((G2, D), jnp.float32),
    mesh=plsc.VectorSubcoreMesh(
        core_axis_name="c", subcore_axis_name="s", num_cores=NC, num_subcores=NS
    ),
    scratch_types=[
        pltpu.VMEM((CHUNK_ROWS, D), jnp.float32),
        pltpu.VMEM((CHUNK_ROWS, D), jnp.float32),
        pltpu.VMEM((CHUNK_G, D), jnp.float32),
        pltpu.VMEM((OUT_W, D), jnp.float32),
        pltpu.VMEM((D,), jnp.float32),
        pltpu.SemaphoreType.DMA,
        pltpu.SemaphoreType.DMA,
    ],
    compiler_params=pltpu.CompilerParams(needs_layout_passes=False),
)
def _sc_kernel(x_hbm, q_hbm, out_hbm, in0, in1, y_buf, out_buf, q_buf, sem0, sem1):
    wid = lax.axis_index("s") * NC + lax.axis_index("c")
    pltpu.sync_copy(q_hbm, q_buf)
    row_base = wid * ROWS_W
    bufs = (in0, in1)
    sems = (sem0, sem1)

    def start(cidx, slot):
        pltpu.async_copy(
            x_hbm.at[pl.ds(row_base + cidx * CHUNK_ROWS, CHUNK_ROWS), :],
            bufs[slot], sems[slot])

    def wait(slot):
        pltpu.make_async_copy(
            x_hbm.at[pl.ds(0, CHUNK_ROWS), :], bufs[slot], sems[slot]).wait()

    start(0, 0)
    start(1, 1)

    def chunk_pair(c2, carry):
        for slot in range(2):
            cidx = c2 * 2 + slot
            wait(slot)
            qv = [q_buf[pl.ds(k * LANES, LANES)] for k in range(KV)]

            def group_body(g2, carry2):
                _attend2(bufs[slot], g2 * 2 * C, qv, y_buf, g2 * 2)
                return carry2

            lax.fori_loop(0, CHUNK_G // 2, group_body, 0)
            _attend2(y_buf, 0, qv, out_buf, cidx * 2)

            @pl.when(c2 * 2 + slot + 2 < NCHUNK)
            def _():
                start(cidx + 2, slot)
        return carry

    lax.fori_loop(0, NCHUNK // 2, chunk_pair, 0)
    pltpu.sync_copy(out_buf, out_hbm.at[pl.ds(wid * OUT_W, OUT_W)])


def kernel(x, segment_ids, q):
    del segment_ids
    return _sc_kernel(x, q)
